# baseline (speedup 1.0000x reference)
import functools

import jax
import jax.numpy as jnp
from jax.experimental import pallas as pl
from jax.experimental.pallas import tpu as pltpu

B, N_MELS, T = 16, 128, 2048
D_MODEL, CODE_DIM, K_CODES = 512, 64, 1024
TQ = T // 4
BOS_TOKEN = K_CODES
EOS_TOKEN = K_CODES + 1
PAD_TOKEN = K_CODES + 2


def _encoder_body(x_ref, w1_ref, b1_ref, w2_ref, b2_ref, wp_ref, bp_ref,
                  cb_ref, tok_ref):
    f32 = jnp.float32
    P0 = x_ref[0, 0]
    P1 = x_ref[0, 1]
    P2 = x_ref[0, 2]
    P3 = x_ref[0, 3]
    zc = jnp.zeros((N_MELS, 1), f32)
    P3m = jnp.concatenate([zc, P3[:, :-1]], axis=1)
    P0p = jnp.concatenate([P0[:, 1:], zc], axis=1)

    def mm(a, b):
        return jnp.dot(a, b, preferred_element_type=f32)

    W10, W11, W12, W13 = w1_ref[0], w1_ref[1], w1_ref[2], w1_ref[3]
    e1 = mm(W10, P3m) + mm(W11, P0) + mm(W12, P1) + mm(W13, P2) + b1_ref[...]
    o1 = mm(W10, P1) + mm(W11, P2) + mm(W12, P3) + mm(W13, P0p) + b1_ref[...]
    E1 = jax.nn.gelu(e1)
    O1 = jax.nn.gelu(o1)

    zd = jnp.zeros((D_MODEL, 1), f32)
    O1m = jnp.concatenate([zd, O1[:, :-1]], axis=1)
    E1p = jnp.concatenate([E1[:, 1:], zd], axis=1)
    W20, W21, W22, W23 = w2_ref[0], w2_ref[1], w2_ref[2], w2_ref[3]
    h2 = mm(W20, O1m) + mm(W21, E1) + mm(W22, O1) + mm(W23, E1p) + b2_ref[...]
    h2 = jax.nn.gelu(h2)

    zT = jax.lax.dot_general(wp_ref[...], h2, (((0,), (0,)), ((), ())),
                             preferred_element_type=f32) + bp_ref[...]
    scores = jax.lax.dot_general(cb_ref[...], zT, (((1,), (0,)), ((), ())),
                                 preferred_element_type=f32)
    cb2 = jnp.sum(cb_ref[...] * cb_ref[...], axis=1, keepdims=True)
    d = cb2 - 2.0 * scores
    dmin = jnp.min(d, axis=0, keepdims=True)
    kio = jax.lax.broadcasted_iota(jnp.int32, (K_CODES, TQ), 0)
    idx = jnp.min(jnp.where(d == dmin, kio, K_CODES), axis=0)
    tok_ref[0, 0, :] = idx.astype(jnp.int32)


def _fixup_body(tok_ref, len_ref, out_ref, lens_ref):
    toks = tok_ref[...]
    lens = len_ref[...] // 4
    j = jax.lax.broadcasted_iota(jnp.int32, (B, TQ + 2), 1)
    shifted = jnp.concatenate(
        [jnp.full((B, 1), PAD_TOKEN, jnp.int32), toks,
         jnp.full((B, 1), PAD_TOKEN, jnp.int32)], axis=1)
    p = j - 1
    out = jnp.where(p == lens, EOS_TOKEN,
                    jnp.where(p > lens, PAD_TOKEN, shifted))
    out_ref[...] = jnp.where(j == 0, BOS_TOKEN, out)
    lens_ref[...] = lens + 2


@jax.jit
def kernel(mel_spec, mel_spec_lengths, w1, b1, w2, b2, wp, bp, codebook):
    f32 = jnp.float32
    x4 = jnp.transpose(mel_spec.reshape(B, N_MELS, TQ, 4), (0, 3, 1, 2))
    w1p = jnp.transpose(w1, (2, 0, 1))
    w2p = jnp.transpose(w2, (2, 0, 1))
    b1c = b1[:, None]
    b2c = b2[:, None]
    bpc = bp[:, None]

    tok_raw = pl.pallas_call(
        _encoder_body,
        grid=(B,),
        in_specs=[
            pl.BlockSpec((1, 4, N_MELS, TQ), lambda b: (b, 0, 0, 0)),
            pl.BlockSpec((4, D_MODEL, N_MELS), lambda b: (0, 0, 0)),
            pl.BlockSpec((D_MODEL, 1), lambda b: (0, 0)),
            pl.BlockSpec((4, D_MODEL, D_MODEL), lambda b: (0, 0, 0)),
            pl.BlockSpec((D_MODEL, 1), lambda b: (0, 0)),
            pl.BlockSpec((D_MODEL, CODE_DIM), lambda b: (0, 0)),
            pl.BlockSpec((CODE_DIM, 1), lambda b: (0, 0)),
            pl.BlockSpec((K_CODES, CODE_DIM), lambda b: (0, 0)),
        ],
        out_specs=pl.BlockSpec((1, 1, TQ), lambda b: (b, 0, 0)),
        out_shape=jax.ShapeDtypeStruct((B, 1, TQ), jnp.int32),
    )(x4, w1p, b1c, w2p, b2c, wp, bpc, codebook)

    tokens, lengths = pl.pallas_call(
        _fixup_body,
        in_specs=[
            pl.BlockSpec((B, TQ), lambda: (0, 0)),
            pl.BlockSpec((B, 1), lambda: (0, 0)),
        ],
        out_specs=[
            pl.BlockSpec((B, TQ + 2), lambda: (0, 0)),
            pl.BlockSpec((B, 1), lambda: (0, 0)),
        ],
        out_shape=[
            jax.ShapeDtypeStruct((B, TQ + 2), jnp.int32),
            jax.ShapeDtypeStruct((B, 1), jnp.int32),
        ],
    )(tok_raw.reshape(B, TQ), mel_spec_lengths.astype(jnp.int32)[:, None])

    return tokens, lengths.reshape(B)


# speedup vs baseline: 1.3661x; 1.3661x over previous
import functools

import jax
import jax.numpy as jnp
from jax.experimental import pallas as pl
from jax.experimental.pallas import tpu as pltpu

B, N_MELS, T = 16, 128, 2048
D_MODEL, CODE_DIM, K_CODES = 512, 64, 1024
TQ = T // 4
BOS_TOKEN = K_CODES
EOS_TOKEN = K_CODES + 1
PAD_TOKEN = K_CODES + 2


def _encoder_body(x_ref, w1_ref, b1_ref, w2_ref, b2_ref, wp_ref, bp_ref,
                  cbt_ref, tok_ref, xt_ref, h1_ref):
    f32 = jnp.float32
    xt_ref[...] = x_ref[0].T
    E = xt_ref[0::2, :]
    O = xt_ref[1::2, :]
    zr = jnp.zeros((1, N_MELS), f32)
    Om = jnp.concatenate([zr, O[:-1, :]], axis=0)
    Ep = jnp.concatenate([E[1:, :], zr], axis=0)

    def mm(a, b):
        return jnp.dot(a, b, preferred_element_type=f32)

    h1 = (mm(Om, w1_ref[0]) + mm(E, w1_ref[1]) + mm(O, w1_ref[2])
          + mm(Ep, w1_ref[3]) + b1_ref[...])
    h1 = jax.nn.gelu(h1)
    for c in range(D_MODEL // N_MELS):
        h1_ref[c] = h1[:, c * N_MELS:(c + 1) * N_MELS]
    E1 = jnp.concatenate([h1_ref[c, 0::2, :] for c in range(4)], axis=1)
    O1 = jnp.concatenate([h1_ref[c, 1::2, :] for c in range(4)], axis=1)
    zd = jnp.zeros((1, D_MODEL), f32)
    O1m = jnp.concatenate([zd, O1[:-1, :]], axis=0)
    E1p = jnp.concatenate([E1[1:, :], zd], axis=0)
    h2 = (mm(O1m, w2_ref[0]) + mm(E1, w2_ref[1]) + mm(O1, w2_ref[2])
          + mm(E1p, w2_ref[3]) + b2_ref[...])
    h2 = jax.nn.gelu(h2)

    z = mm(h2, wp_ref[...]) + bp_ref[...]
    scores = mm(z, cbt_ref[...])
    cbt = cbt_ref[...]
    cb2 = jnp.sum(cbt * cbt, axis=0, keepdims=True)
    d = cb2 - 2.0 * scores
    dmin = jnp.min(d, axis=1, keepdims=True)
    kio = jax.lax.broadcasted_iota(jnp.int32, (TQ, K_CODES), 1)
    idx = jnp.min(jnp.where(d == dmin, kio, K_CODES), axis=1)
    tok_ref[0, :, 0] = idx.astype(jnp.int32)


def _fixup_body(tok_ref, len_ref, out_ref, lens_ref):
    toks = tok_ref[...]
    lens = len_ref[...] // 4
    j = jax.lax.broadcasted_iota(jnp.int32, (B, TQ + 2), 1)
    shifted = jnp.concatenate(
        [jnp.full((B, 1), PAD_TOKEN, jnp.int32), toks,
         jnp.full((B, 1), PAD_TOKEN, jnp.int32)], axis=1)
    p = j - 1
    out = jnp.where(p == lens, EOS_TOKEN,
                    jnp.where(p > lens, PAD_TOKEN, shifted))
    out_ref[...] = jnp.where(j == 0, BOS_TOKEN, out)
    lens_ref[...] = lens + 2


@jax.jit
def kernel(mel_spec, mel_spec_lengths, w1, b1, w2, b2, wp, bp, codebook):
    w1t = jnp.transpose(w1, (2, 1, 0))
    w2t = jnp.transpose(w2, (2, 1, 0))
    b1r = b1[None, :]
    b2r = b2[None, :]
    bpr = bp[None, :]
    cbt = codebook.T

    tok_raw = pl.pallas_call(
        _encoder_body,
        grid=(B,),
        in_specs=[
            pl.BlockSpec((1, N_MELS, T), lambda b: (b, 0, 0)),
            pl.BlockSpec((4, N_MELS, D_MODEL), lambda b: (0, 0, 0)),
            pl.BlockSpec((1, D_MODEL), lambda b: (0, 0)),
            pl.BlockSpec((4, D_MODEL, D_MODEL), lambda b: (0, 0, 0)),
            pl.BlockSpec((1, D_MODEL), lambda b: (0, 0)),
            pl.BlockSpec((D_MODEL, CODE_DIM), lambda b: (0, 0)),
            pl.BlockSpec((1, CODE_DIM), lambda b: (0, 0)),
            pl.BlockSpec((CODE_DIM, K_CODES), lambda b: (0, 0)),
        ],
        out_specs=pl.BlockSpec((1, TQ, 1), lambda b: (b, 0, 0)),
        out_shape=jax.ShapeDtypeStruct((B, TQ, 1), jnp.int32),
        scratch_shapes=[
            pltpu.VMEM((T, N_MELS), jnp.float32),
            pltpu.VMEM((4, T // 2, N_MELS), jnp.float32),
        ],
    )(mel_spec, w1t, b1r, w2t, b2r, wp, bpr, cbt)

    tokens, lengths = pl.pallas_call(
        _fixup_body,
        in_specs=[
            pl.BlockSpec((B, TQ), lambda: (0, 0)),
            pl.BlockSpec((B, 1), lambda: (0, 0)),
        ],
        out_specs=[
            pl.BlockSpec((B, TQ + 2), lambda: (0, 0)),
            pl.BlockSpec((B, 1), lambda: (0, 0)),
        ],
        out_shape=[
            jax.ShapeDtypeStruct((B, TQ + 2), jnp.int32),
            jax.ShapeDtypeStruct((B, 1), jnp.int32),
        ],
    )(tok_raw.reshape(B, TQ), mel_spec_lengths.astype(jnp.int32)[:, None])

    return tokens, lengths.reshape(B)
